# baseline (speedup 1.0000x reference)
import functools

import jax
import jax.numpy as jnp
from jax import lax
from jax.experimental import pallas as pl
from jax.experimental.pallas import tpu as pltpu
from jax.experimental.pallas import tpu_sc as plsc

N = 100000
D = 128
LANES = 16
NCORES = 2
NSUB = 16
NW = NCORES * NSUB
CHUNK = 400
NCHUNK = N // CHUNK
GROUPS = CHUNK // LANES
GROUP_UNROLL = 5


def _sc_body(p_hbm, tab_hbm, out_hbm, p_v0, p_v1, o_v0, o_v1, t_v,
             psem, osem):
    w = lax.axis_index("s") * NCORES + lax.axis_index("c")
    nc = (NCHUNK - w + NW - 1) // NW

    pltpu.sync_copy(tab_hbm.at[pl.ds(0, 2)], t_v)
    t0s = [t_v[0, pl.ds(16 * j, 16)] for j in range(D // LANES)]
    dls = [t_v[1, pl.ds(16 * j, 16)] - t0s[j] for j in range(D // LANES)]

    def base(k):
        return (w + k * NW) * CHUNK

    def do_chunk(k, pb, pb_next, ob):
        pltpu.make_async_copy(
            p_hbm.at[pl.ds(base(k), CHUNK)], pb, psem).wait()

        @pl.when(k + 1 < nc)
        def _():
            pltpu.async_copy(
                p_hbm.at[pl.ds(base(k + 1), CHUNK)], pb_next, psem)

        @pl.when(k >= 2)
        def _():
            pltpu.make_async_copy(
                ob, out_hbm.at[pl.ds(base(k), CHUNK)], osem).wait()

        def group_body(g, c):
            for gg in range(GROUP_UNROLL):
                gb = LANES * (GROUP_UNROLL * g + gg)
                pv = pb[pl.ds(gb, LANES)]
                for r in range(LANES):
                    pi = pv[r]
                    i = gb + r
                    for j in range(D // LANES):
                        ob[i, pl.ds(16 * j, 16)] = t0s[j] + pi * dls[j]
            return c

        lax.fori_loop(0, GROUPS // GROUP_UNROLL, group_body, 0)
        pltpu.async_copy(ob, out_hbm.at[pl.ds(base(k), CHUNK)], osem)

    pltpu.async_copy(p_hbm.at[pl.ds(base(0), CHUNK)], p_v0, psem)

    def pair_body(q, carry):
        do_chunk(2 * q, p_v0, p_v1, o_v0)
        do_chunk(2 * q + 1, p_v1, p_v0, o_v1)
        return carry

    lax.fori_loop(0, nc // 2, pair_body, 0)

    @pl.when(lax.rem(nc, 2) == 1)
    def _():
        do_chunk(nc - 1, p_v0, p_v1, o_v0)

    pltpu.make_async_copy(o_v0, out_hbm.at[pl.ds(0, CHUNK)], osem).wait()
    pltpu.make_async_copy(o_v1, out_hbm.at[pl.ds(0, CHUNK)], osem).wait()


def kernel(label_probs, table):
    mesh = plsc.VectorSubcoreMesh(core_axis_name="c", subcore_axis_name="s")
    f = functools.partial(
        pl.kernel,
        out_type=jax.ShapeDtypeStruct((N, D), jnp.float32),
        mesh=mesh,
        scratch_types=[
            pltpu.VMEM((CHUNK,), jnp.float32),
            pltpu.VMEM((CHUNK,), jnp.float32),
            pltpu.VMEM((CHUNK, D), jnp.float32),
            pltpu.VMEM((CHUNK, D), jnp.float32),
            pltpu.VMEM((2, D), jnp.float32),
            pltpu.SemaphoreType.DMA,
            pltpu.SemaphoreType.DMA,
        ],
    )(_sc_body)
    return f(label_probs, table)


# speedup vs baseline: 1.0917x; 1.0917x over previous
import functools

import jax
import jax.numpy as jnp
from jax import lax
from jax.experimental import pallas as pl
from jax.experimental.pallas import tpu as pltpu
from jax.experimental.pallas import tpu_sc as plsc

N = 100000
D = 128
LANES = 16
NCORES = 2
NSUB = 16
NW = NCORES * NSUB
CHUNK = 400
NCHUNK = N // CHUNK
GROUPS = CHUNK // LANES


def _sc_body(p_hbm, tab_hbm, out_hbm, p_v, o_v, t_v, psem, osem):
    w = lax.axis_index("s") * NCORES + lax.axis_index("c")
    nc = (NCHUNK - w + NW - 1) // NW

    pltpu.sync_copy(tab_hbm.at[pl.ds(0, 2)], t_v)
    t0s = [t_v[0, pl.ds(16 * j, 16)] for j in range(D // LANES)]
    dls = [t_v[1, pl.ds(16 * j, 16)] - t0s[j] for j in range(D // LANES)]

    def base(k):
        return (w + k * NW) * CHUNK

    pltpu.async_copy(p_hbm.at[pl.ds(base(0), CHUNK)], p_v.at[pl.ds(0, CHUNK)],
                     psem)

    def chunk_body(k, carry):
        off = lax.rem(k, 2) * CHUNK
        pb = p_v.at[pl.ds(off, CHUNK)]
        ob = o_v.at[pl.ds(off, CHUNK)]

        pltpu.make_async_copy(
            p_hbm.at[pl.ds(base(k), CHUNK)], pb, psem).wait()

        @pl.when(k + 1 < nc)
        def _():
            pltpu.async_copy(
                p_hbm.at[pl.ds(base(k + 1), CHUNK)],
                p_v.at[pl.ds(CHUNK - off, CHUNK)], psem)

        @pl.when(k >= 2)
        def _():
            pltpu.make_async_copy(
                ob, out_hbm.at[pl.ds(base(k), CHUNK)], osem).wait()

        def group_body(g, c):
            pv = pb[pl.ds(LANES * g, LANES)]
            for r in range(LANES):
                pi = pv[r]
                i = LANES * g + r
                for j in range(D // LANES):
                    ob[i, pl.ds(16 * j, 16)] = t0s[j] + pi * dls[j]
            return c

        lax.fori_loop(0, GROUPS, group_body, 0)
        pltpu.async_copy(ob, out_hbm.at[pl.ds(base(k), CHUNK)], osem)
        return carry

    lax.fori_loop(0, nc, chunk_body, 0)

    ob0 = o_v.at[pl.ds(0, CHUNK)]
    pltpu.make_async_copy(ob0, out_hbm.at[pl.ds(0, CHUNK)], osem).wait()
    pltpu.make_async_copy(ob0, out_hbm.at[pl.ds(0, CHUNK)], osem).wait()


def kernel(label_probs, table):
    mesh = plsc.VectorSubcoreMesh(core_axis_name="c", subcore_axis_name="s")
    f = functools.partial(
        pl.kernel,
        out_type=jax.ShapeDtypeStruct((N, D), jnp.float32),
        mesh=mesh,
        scratch_types=[
            pltpu.VMEM((2 * CHUNK,), jnp.float32),
            pltpu.VMEM((2 * CHUNK, D), jnp.float32),
            pltpu.VMEM((2, D), jnp.float32),
            pltpu.SemaphoreType.DMA,
            pltpu.SemaphoreType.DMA,
        ],
    )(_sc_body)
    return f(label_probs, table)
